# baseline (speedup 1.0000x reference)
import jax
import jax.numpy as jnp
from jax import lax
from jax.experimental import pallas as pl
from jax.experimental.pallas import tpu as pltpu
from jax.experimental.pallas import tpu_sc as plsc

N = 10000
E = 320000
DIN = 128
EMBD = 64
H = 256
HH = 128
VOCAB = 1000
OUTD = 16
G = 64
NS = 16

CH = 128
NCHUNK = E // CH
UNR = 12
FULL = (NCHUNK // NS) // UNR
REM = NCHUNK - FULL * UNR * NS

NIDXCH = 80
NPADN = NIDXCH * CH

RPT = 624
TAILOFF = RPT * NS
TAIL = N - TAILOFF

BLK = 1000
GRID = N // BLK

_f32 = jnp.float32
_i32 = jnp.int32

_mesh = plsc.VectorSubcoreMesh(core_axis_name="c", subcore_axis_name="s")


NHIST = 10240
HSL = NHIST // NS


def _sc_pre_body(colch, nidx2d, embp, zeros1d,
                 deg, degb, embv,
                 idxc0, idxc1,
                 idxe_v, erows, hist, mslab, msum,
                 hshared,
                 semc0, semc1, semg0, semg1, sem_x):
    c = lax.axis_index("c")
    s = lax.axis_index("s")

    pltpu.sync_copy(zeros1d, hist)

    @pl.when(c == 1)
    def _emb():
        s5 = s * (NIDXCH // NS)
        semg = [semg0, semg1]
        for j in range(NIDXCH // NS):
            pltpu.async_copy(nidx2d.at[s5 + j], idxe_v, sem_x).wait()
            pltpu.async_copy(embp.at[idxe_v], erows, semg[0]).wait()
            pltpu.async_copy(
                erows, embv.at[pl.ds((s5 + j) * CH, CH)], semg[1]).wait()

    cbuf = [idxc0, idxc1]
    semc = [semc0, semc1]
    base_d = c * (NCHUNK // 2)
    UD = 6
    ones16 = jnp.ones((16,), _f32)

    def _hist_chunk(buf):
        for kk in range(CH // 16):
            idx16 = buf[pl.ds(kk * 16, 16)]
            plsc.addupdate_scatter(hist, [idx16], ones16)

    def dbody(i, _):
        def t(j):
            return base_d + s + NS * (UD * i + j)

        ldc = {}
        for j in range(2):
            ldc[j] = pltpu.async_copy(colch.at[t(j)], cbuf[j], semc[j])
        for j in range(UD):
            ldc[j].wait()
            if j + 2 < UD:
                ldc[j + 2] = pltpu.async_copy(
                    colch.at[t(j + 2)], cbuf[j % 2], semc[j % 2])
            _hist_chunk(cbuf[j % 2])
        return 0

    lax.fori_loop(0, (NCHUNK // 2 - 2) // (UD * NS), dbody, 0)

    @pl.when(s < 2)
    def _extra():
        textra = base_d + NCHUNK // 2 - 2 + s
        pltpu.async_copy(colch.at[textra], idxc0, semc0).wait()
        _hist_chunk(idxc0)

    pltpu.sync_copy(hist, hshared.at[s])
    plsc.subcore_barrier()
    sl = pl.ds(s * HSL, HSL)
    pltpu.sync_copy(hshared.at[0, sl], msum)
    for tt in range(1, NS):
        pltpu.sync_copy(hshared.at[tt, sl], mslab)
        for k in range(HSL // 16):
            w = pl.ds(k * 16, 16)
            msum[w] = msum[w] + mslab[w]

    @pl.when(c == 0)
    def _outa():
        pltpu.sync_copy(msum, deg.at[sl])

    @pl.when(c == 1)
    def _outb():
        pltpu.sync_copy(msum, degb.at[sl])


_sc_pre = pl.kernel(
    _sc_pre_body,
    out_type=(jax.ShapeDtypeStruct((NHIST,), _f32),
              jax.ShapeDtypeStruct((NHIST,), _f32),
              jax.ShapeDtypeStruct((NPADN, HH), _f32)),
    mesh=_mesh,
    scratch_types=[
        pltpu.VMEM((CH,), _i32),
        pltpu.VMEM((CH,), _i32),
        pltpu.VMEM((CH,), _i32),
        pltpu.VMEM((CH, HH), _f32),
        pltpu.VMEM((NHIST,), _f32),
        pltpu.VMEM((HSL,), _f32),
        pltpu.VMEM((HSL,), _f32),
        pltpu.VMEM_SHARED((NS, NHIST), _f32),
    ] + [pltpu.SemaphoreType.DMA] * 5,
    compiler_params=pltpu.CompilerParams(needs_layout_passes=False),
)


def _sc_scatter_body(mA, mB, rowch, colch, zeros128,
                     sA, sB,
                     idxr0, idxr1, idxr2, idxr3,
                     idxc0, idxc1, idxc2, idxc3,
                     rows, acc,
                     semr0, semr1, semr2, semr3,
                     semc0, semc1, semc2, semc3,
                     semg0, semg1, semg2,
                     sems0, sems1, sems2, sems3):
    c = lax.axis_index("c")
    s = lax.axis_index("s")
    slab = pl.ds(s * RPT, RPT)
    tslab = pl.ds(TAILOFF, TAIL)

    pltpu.sync_copy(zeros128.at[slab], acc.at[slab])

    @pl.when(s == NS - 1)
    def _init_tail():
        pltpu.sync_copy(zeros128.at[tslab], acc.at[tslab])

    plsc.subcore_barrier()

    rbuf = [idxr0, idxr1, idxr2, idxr3]
    cbuf = [idxc0, idxc1, idxc2, idxc3]
    semr = [semr0, semr1, semr2, semr3]
    semc = [semc0, semc1, semc2, semc3]
    semg = [semg0, semg1, semg2]
    sems = [sems0, sems1, sems2, sems3]

    def _pipeline(tbl):
        def body(i, _):
            def t(j):
                return s + NS * (UNR * i + j)

            ldr = {}
            ldc = {}
            g = {}
            sc = {}
            for j in range(3):
                ldr[j] = pltpu.async_copy(rowch.at[t(j)], rbuf[j], semr[j])
            for j in range(2):
                ldc[j] = pltpu.async_copy(colch.at[t(j)], cbuf[j], semc[j])
            for j in range(2):
                ldr[j].wait()
                g[j] = pltpu.async_copy(
                    tbl.at[rbuf[j]], rows.at[j % 3], semg[j % 3])
            for j in range(UNR):
                g[j].wait()
                ldc[j].wait()
                if j + 2 < UNR:
                    ldr[j + 2].wait()
                    g[j + 2] = pltpu.async_copy(
                        tbl.at[rbuf[(j + 2) % 4]], rows.at[(j + 2) % 3],
                        semg[(j + 2) % 3])
                pltpu.sync_copy(rows.at[j % 3], acc.at[cbuf[j % 4]], add=True)
                if j + 3 < UNR:
                    ldr[j + 3] = pltpu.async_copy(
                        rowch.at[t(j + 3)], rbuf[(j + 3) % 4],
                        semr[(j + 3) % 4])
                if j + 2 < UNR:
                    ldc[j + 2] = pltpu.async_copy(
                        colch.at[t(j + 2)], cbuf[(j + 2) % 4],
                        semc[(j + 2) % 4])
            return 0

        lax.fori_loop(0, FULL, body, 0)

        @pl.when(s < REM)
        def _extra():
            textra = FULL * UNR * NS + s
            pltpu.async_copy(rowch.at[textra], rbuf[0], semr[0]).wait()
            pltpu.async_copy(colch.at[textra], cbuf[0], semc[0]).wait()
            pltpu.async_copy(tbl.at[rbuf[0]], rows.at[0], semg[0]).wait()
            pltpu.sync_copy(rows.at[0], acc.at[cbuf[0]], add=True)

    @pl.when(c == 0)
    def _pa():
        _pipeline(mA)

    @pl.when(c == 1)
    def _pb():
        _pipeline(mB)

    plsc.subcore_barrier()

    @pl.when(c == 0)
    def _oa():
        pltpu.sync_copy(acc.at[slab], sA.at[slab])

        @pl.when(s == NS - 1)
        def _oa_tail():
            pltpu.sync_copy(acc.at[tslab], sA.at[tslab])

    @pl.when(c == 1)
    def _ob():
        pltpu.sync_copy(acc.at[slab], sB.at[slab])

        @pl.when(s == NS - 1)
        def _ob_tail():
            pltpu.sync_copy(acc.at[tslab], sB.at[tslab])


_sc_scatter = pl.kernel(
    _sc_scatter_body,
    out_type=(jax.ShapeDtypeStruct((N, HH), _f32),
              jax.ShapeDtypeStruct((N, HH), _f32)),
    mesh=_mesh,
    scratch_types=[
        pltpu.VMEM((CH,), _i32),
        pltpu.VMEM((CH,), _i32),
        pltpu.VMEM((CH,), _i32),
        pltpu.VMEM((CH,), _i32),
        pltpu.VMEM((CH,), _i32),
        pltpu.VMEM((CH,), _i32),
        pltpu.VMEM((CH,), _i32),
        pltpu.VMEM((CH,), _i32),
        pltpu.VMEM((3, CH, HH), _f32),
        pltpu.VMEM_SHARED((N, HH), _f32),
    ] + [pltpu.SemaphoreType.DMA] * 15,
)


def _dinv_of(degr):
    return lax.rsqrt(degr[:, 0:1] + 1.0)


def _gelu(v):
    return 0.5 * v * (1.0 + lax.erf(v * 0.7071067811865476))


def _tc_layer1_body(featsr, embvr, w1ar, w1br, degr, oa, ob):
    dinv = _dinv_of(degr[...])
    m = jnp.dot(featsr[...], w1ar[...], preferred_element_type=_f32)
    m = m + jnp.dot(embvr[...], w1br[...], preferred_element_type=_f32)
    mt = m * dinv
    oa[...] = mt[:, :HH]
    ob[...] = mt[:, HH:]


_tc_layer1 = pl.pallas_call(
    _tc_layer1_body,
    grid=(GRID,),
    in_specs=[
        pl.BlockSpec((BLK, DIN), lambda i: (i, 0)),
        pl.BlockSpec((BLK, EMBD), lambda i: (i, 0)),
        pl.BlockSpec((DIN, H), lambda i: (0, 0)),
        pl.BlockSpec((EMBD, H), lambda i: (0, 0)),
        pl.BlockSpec((BLK, 8), lambda i: (i, 0)),
    ],
    out_specs=[
        pl.BlockSpec((BLK, HH), lambda i: (i, 0)),
        pl.BlockSpec((BLK, HH), lambda i: (i, 0)),
    ],
    out_shape=[
        jax.ShapeDtypeStruct((N, HH), _f32),
        jax.ShapeDtypeStruct((N, HH), _f32),
    ],
)


def _tc_mid_body(sar, sbr, mar, mbr, degr, br, wr, oa, ob):
    dinv = _dinv_of(degr[...])
    h = jnp.concatenate([sar[...] + mar[...], sbr[...] + mbr[...]], axis=1)
    h = _gelu(h * dinv + br[...])
    mt = jnp.dot(h, wr[...], preferred_element_type=_f32) * dinv
    oa[...] = mt[:, :HH]
    ob[...] = mt[:, HH:]


_tc_mid = pl.pallas_call(
    _tc_mid_body,
    grid=(GRID,),
    in_specs=[
        pl.BlockSpec((BLK, HH), lambda i: (i, 0)),
        pl.BlockSpec((BLK, HH), lambda i: (i, 0)),
        pl.BlockSpec((BLK, HH), lambda i: (i, 0)),
        pl.BlockSpec((BLK, HH), lambda i: (i, 0)),
        pl.BlockSpec((BLK, 8), lambda i: (i, 0)),
        pl.BlockSpec((1, H), lambda i: (0, 0)),
        pl.BlockSpec((H, H), lambda i: (0, 0)),
    ],
    out_specs=[
        pl.BlockSpec((BLK, HH), lambda i: (i, 0)),
        pl.BlockSpec((BLK, HH), lambda i: (i, 0)),
    ],
    out_shape=[
        jax.ShapeDtypeStruct((N, HH), _f32),
        jax.ShapeDtypeStruct((N, HH), _f32),
    ],
)


def _tc_final_body(sar, sbr, mar, mbr, degr, br, batchr, wfr, bfr, outr, gacc):
    i = pl.program_id(0)
    dinv = _dinv_of(degr[...])
    h = jnp.concatenate([sar[...] + mar[...], sbr[...] + mbr[...]], axis=1)
    h = _gelu(h * dinv + br[...])
    bvec = batchr[0, 0, :]
    onehot = (bvec[:, None] ==
              lax.broadcasted_iota(_i32, (BLK, G), 1)).astype(_f32)
    contrib = lax.dot_general(onehot, h, (((0,), (0,)), ((), ())),
                              preferred_element_type=_f32)

    @pl.when(i == 0)
    def _first():
        gacc[...] = contrib

    @pl.when(i > 0)
    def _rest():
        gacc[...] = gacc[...] + contrib

    @pl.when(i == GRID - 1)
    def _emit():
        outr[...] = _gelu(
            jnp.dot(gacc[...], wfr[...], preferred_element_type=_f32) + bfr[...])


_tc_final = pl.pallas_call(
    _tc_final_body,
    grid=(GRID,),
    in_specs=[
        pl.BlockSpec((BLK, HH), lambda i: (i, 0)),
        pl.BlockSpec((BLK, HH), lambda i: (i, 0)),
        pl.BlockSpec((BLK, HH), lambda i: (i, 0)),
        pl.BlockSpec((BLK, HH), lambda i: (i, 0)),
        pl.BlockSpec((BLK, 8), lambda i: (i, 0)),
        pl.BlockSpec((1, H), lambda i: (0, 0)),
        pl.BlockSpec((1, 1, BLK), lambda i: (i, 0, 0)),
        pl.BlockSpec((H, 128), lambda i: (0, 0)),
        pl.BlockSpec((1, 128), lambda i: (0, 0)),
    ],
    out_specs=pl.BlockSpec((G, 128), lambda i: (0, 0)),
    out_shape=jax.ShapeDtypeStruct((G, 128), _f32),
    scratch_shapes=[pltpu.VMEM((G, H), _f32)],
)


@jax.jit
def kernel(x, edge_index, batch, emb, W1, b1, W2, b2, W3, b3, Wf, bf):
    row2d = edge_index[0].astype(_i32).reshape(NCHUNK, CH)
    col2d = edge_index[1].astype(_i32).reshape(NCHUNK, CH)
    nidx = x[:, -1].astype(_i32)
    nidx2d = jnp.concatenate(
        [nidx, jnp.zeros((NPADN - N,), _i32)]).reshape(NIDXCH, CH)
    feats = x[:, :DIN]
    batch3 = batch.astype(_i32).reshape(GRID, 1, BLK)

    zeros128 = jnp.zeros((N, HH), _f32)
    embp = jnp.pad(emb, ((0, 0), (0, HH - EMBD)))

    dega, degb, embvp = _sc_pre(col2d, nidx2d, embp, jnp.zeros((NHIST,), _f32))
    embv = embvp[:N, :EMBD]
    deg8 = (dega[:N] + degb[:N])[:, None] * jnp.ones((1, 8), _f32)

    m1a, m1b = _tc_layer1(feats, embv, W1[:DIN], W1[DIN:], deg8)
    s1a, s1b = _sc_scatter(m1a, m1b, row2d, col2d, zeros128)
    m2a, m2b = _tc_mid(s1a, s1b, m1a, m1b, deg8, b1.reshape(1, H), W2)
    s2a, s2b = _sc_scatter(m2a, m2b, row2d, col2d, zeros128)
    m3a, m3b = _tc_mid(s2a, s2b, m2a, m2b, deg8, b2.reshape(1, H), W3)
    s3a, s3b = _sc_scatter(m3a, m3b, row2d, col2d, zeros128)

    wfp = jnp.pad(Wf, ((0, 0), (0, 128 - OUTD)))
    bfp = jnp.pad(bf, (0, 128 - OUTD)).reshape(1, 128)
    out128 = _tc_final(s3a, s3b, m3a, m3b, deg8, b3.reshape(1, H),
                       batch3, wfp, bfp)
    return out128[:, :OUTD]


# speedup vs baseline: 1.0329x; 1.0329x over previous
import jax
import jax.numpy as jnp
from jax import lax
from jax.experimental import pallas as pl
from jax.experimental.pallas import tpu as pltpu
from jax.experimental.pallas import tpu_sc as plsc

N = 10000
E = 320000
DIN = 128
EMBD = 64
H = 256
HH = 128
VOCAB = 1000
OUTD = 16
G = 64
NS = 16

CH = 128
NCHUNK = E // CH
UNR = 12
FULL = (NCHUNK // NS) // UNR
REM = NCHUNK - FULL * UNR * NS

NIDXCH = 80
NPADN = NIDXCH * CH

RPT = 624
TAILOFF = RPT * NS
TAIL = N - TAILOFF

BLK = 1000
GRID = N // BLK

_f32 = jnp.float32
_i32 = jnp.int32

_mesh = plsc.VectorSubcoreMesh(core_axis_name="c", subcore_axis_name="s")


NHIST = 10240
HSL = NHIST // NS


def _sc_pre_body(colch, nidx2d, embp, zeros1d,
                 deg, degb, embv,
                 idxc0, idxc1,
                 idxe_v, erows, hist, mslab, msum,
                 hshared,
                 semc0, semc1, semg0, semg1, sem_x):
    c = lax.axis_index("c")
    s = lax.axis_index("s")

    pltpu.sync_copy(zeros1d, hist)

    @pl.when(c == 1)
    def _emb():
        s5 = s * (NIDXCH // NS)
        semg = [semg0, semg1]
        for j in range(NIDXCH // NS):
            pltpu.async_copy(nidx2d.at[s5 + j], idxe_v, sem_x).wait()
            pltpu.async_copy(embp.at[idxe_v], erows, semg[0]).wait()
            pltpu.async_copy(
                erows, embv.at[pl.ds((s5 + j) * CH, CH)], semg[1]).wait()

    cbuf = [idxc0, idxc1]
    semc = [semc0, semc1]
    base_d = c * (NCHUNK // 2)
    UD = 6
    ones16 = jnp.ones((16,), _f32)

    def _hist_chunk(buf):
        for kk in range(CH // 16):
            idx16 = buf[pl.ds(kk * 16, 16)]
            plsc.addupdate_scatter(hist, [idx16], ones16)

    def dbody(i, _):
        def t(j):
            return base_d + s + NS * (UD * i + j)

        ldc = {}
        for j in range(2):
            ldc[j] = pltpu.async_copy(colch.at[t(j)], cbuf[j], semc[j])
        for j in range(UD):
            ldc[j].wait()
            if j + 2 < UD:
                ldc[j + 2] = pltpu.async_copy(
                    colch.at[t(j + 2)], cbuf[j % 2], semc[j % 2])
            _hist_chunk(cbuf[j % 2])
        return 0

    lax.fori_loop(0, (NCHUNK // 2 - 2) // (UD * NS), dbody, 0)

    @pl.when(s < 2)
    def _extra():
        textra = base_d + NCHUNK // 2 - 2 + s
        pltpu.async_copy(colch.at[textra], idxc0, semc0).wait()
        _hist_chunk(idxc0)

    pltpu.sync_copy(hist, hshared.at[s])
    plsc.subcore_barrier()
    sl = pl.ds(s * HSL, HSL)
    pltpu.sync_copy(hshared.at[0, sl], msum)
    for tt in range(1, NS):
        pltpu.sync_copy(hshared.at[tt, sl], mslab)
        for k in range(HSL // 16):
            w = pl.ds(k * 16, 16)
            msum[w] = msum[w] + mslab[w]

    @pl.when(c == 0)
    def _outa():
        pltpu.sync_copy(msum, deg.at[sl])

    @pl.when(c == 1)
    def _outb():
        pltpu.sync_copy(msum, degb.at[sl])


_sc_pre = pl.kernel(
    _sc_pre_body,
    out_type=(jax.ShapeDtypeStruct((NHIST,), _f32),
              jax.ShapeDtypeStruct((NHIST,), _f32),
              jax.ShapeDtypeStruct((NPADN, HH), _f32)),
    mesh=_mesh,
    scratch_types=[
        pltpu.VMEM((CH,), _i32),
        pltpu.VMEM((CH,), _i32),
        pltpu.VMEM((CH,), _i32),
        pltpu.VMEM((CH, HH), _f32),
        pltpu.VMEM((NHIST,), _f32),
        pltpu.VMEM((HSL,), _f32),
        pltpu.VMEM((HSL,), _f32),
        pltpu.VMEM_SHARED((NS, NHIST), _f32),
    ] + [pltpu.SemaphoreType.DMA] * 5,
    compiler_params=pltpu.CompilerParams(needs_layout_passes=False),
)


def _sc_scatter_body(mA, mB, rowch, colch, zeros128,
                     sA, sB,
                     idxr0, idxr1, idxr2, idxr3,
                     idxc0, idxc1, idxc2, idxc3,
                     rows, acc,
                     semr0, semr1, semr2, semr3,
                     semc0, semc1, semc2, semc3,
                     semg0, semg1, semg2,
                     sems0, sems1, sems2, sems3):
    c = lax.axis_index("c")
    s = lax.axis_index("s")
    slab = pl.ds(s * RPT, RPT)
    tslab = pl.ds(TAILOFF, TAIL)

    pltpu.sync_copy(zeros128.at[slab], acc.at[slab])

    @pl.when(s == NS - 1)
    def _init_tail():
        pltpu.sync_copy(zeros128.at[tslab], acc.at[tslab])

    plsc.subcore_barrier()

    rbuf = [idxr0, idxr1, idxr2, idxr3]
    cbuf = [idxc0, idxc1, idxc2, idxc3]
    semr = [semr0, semr1, semr2, semr3]
    semc = [semc0, semc1, semc2, semc3]
    semg = [semg0, semg1, semg2]
    sems = [sems0, sems1, sems2, sems3]

    def _pipeline(tbl):
        def body(i, _):
            def t(j):
                return s + NS * (UNR * i + j)

            ldr = {}
            ldc = {}
            g = {}
            sc = {}
            for j in range(3):
                ldr[j] = pltpu.async_copy(rowch.at[t(j)], rbuf[j], semr[j])
            for j in range(2):
                ldc[j] = pltpu.async_copy(colch.at[t(j)], cbuf[j], semc[j])
            for j in range(2):
                ldr[j].wait()
                g[j] = pltpu.async_copy(
                    tbl.at[rbuf[j]], rows.at[j % 3], semg[j % 3])
            for j in range(UNR):
                g[j].wait()
                ldc[j].wait()
                pltpu.sync_copy(rows.at[j % 3], acc.at[cbuf[j % 4]], add=True)
                if j + 2 < UNR:
                    ldr[j + 2].wait()
                    g[j + 2] = pltpu.async_copy(
                        tbl.at[rbuf[(j + 2) % 4]], rows.at[(j + 2) % 3],
                        semg[(j + 2) % 3])
                if j + 3 < UNR:
                    ldr[j + 3] = pltpu.async_copy(
                        rowch.at[t(j + 3)], rbuf[(j + 3) % 4],
                        semr[(j + 3) % 4])
                if j + 2 < UNR:
                    ldc[j + 2] = pltpu.async_copy(
                        colch.at[t(j + 2)], cbuf[(j + 2) % 4],
                        semc[(j + 2) % 4])
            return 0

        lax.fori_loop(0, FULL, body, 0)

        @pl.when(s < REM)
        def _extra():
            textra = FULL * UNR * NS + s
            pltpu.async_copy(rowch.at[textra], rbuf[0], semr[0]).wait()
            pltpu.async_copy(colch.at[textra], cbuf[0], semc[0]).wait()
            pltpu.async_copy(tbl.at[rbuf[0]], rows.at[0], semg[0]).wait()
            pltpu.sync_copy(rows.at[0], acc.at[cbuf[0]], add=True)

    @pl.when(c == 0)
    def _pa():
        _pipeline(mA)

    @pl.when(c == 1)
    def _pb():
        _pipeline(mB)

    plsc.subcore_barrier()

    @pl.when(c == 0)
    def _oa():
        pltpu.sync_copy(acc.at[slab], sA.at[slab])

        @pl.when(s == NS - 1)
        def _oa_tail():
            pltpu.sync_copy(acc.at[tslab], sA.at[tslab])

    @pl.when(c == 1)
    def _ob():
        pltpu.sync_copy(acc.at[slab], sB.at[slab])

        @pl.when(s == NS - 1)
        def _ob_tail():
            pltpu.sync_copy(acc.at[tslab], sB.at[tslab])


_sc_scatter = pl.kernel(
    _sc_scatter_body,
    out_type=(jax.ShapeDtypeStruct((N, HH), _f32),
              jax.ShapeDtypeStruct((N, HH), _f32)),
    mesh=_mesh,
    scratch_types=[
        pltpu.VMEM((CH,), _i32),
        pltpu.VMEM((CH,), _i32),
        pltpu.VMEM((CH,), _i32),
        pltpu.VMEM((CH,), _i32),
        pltpu.VMEM((CH,), _i32),
        pltpu.VMEM((CH,), _i32),
        pltpu.VMEM((CH,), _i32),
        pltpu.VMEM((CH,), _i32),
        pltpu.VMEM((3, CH, HH), _f32),
        pltpu.VMEM_SHARED((N, HH), _f32),
    ] + [pltpu.SemaphoreType.DMA] * 15,
)


def _dinv_of(degr):
    return lax.rsqrt(degr[:, 0:1] + 1.0)


def _gelu(v):
    return 0.5 * v * (1.0 + lax.erf(v * 0.7071067811865476))


def _tc_layer1_body(featsr, embvr, w1ar, w1br, degr, oa, ob):
    dinv = _dinv_of(degr[...])
    m = jnp.dot(featsr[...], w1ar[...], preferred_element_type=_f32)
    m = m + jnp.dot(embvr[...], w1br[...], preferred_element_type=_f32)
    mt = m * dinv
    oa[...] = mt[:, :HH]
    ob[...] = mt[:, HH:]


_tc_layer1 = pl.pallas_call(
    _tc_layer1_body,
    grid=(GRID,),
    in_specs=[
        pl.BlockSpec((BLK, DIN), lambda i: (i, 0)),
        pl.BlockSpec((BLK, EMBD), lambda i: (i, 0)),
        pl.BlockSpec((DIN, H), lambda i: (0, 0)),
        pl.BlockSpec((EMBD, H), lambda i: (0, 0)),
        pl.BlockSpec((BLK, 8), lambda i: (i, 0)),
    ],
    out_specs=[
        pl.BlockSpec((BLK, HH), lambda i: (i, 0)),
        pl.BlockSpec((BLK, HH), lambda i: (i, 0)),
    ],
    out_shape=[
        jax.ShapeDtypeStruct((N, HH), _f32),
        jax.ShapeDtypeStruct((N, HH), _f32),
    ],
)


def _tc_mid_body(sar, sbr, mar, mbr, degr, br, wr, oa, ob):
    dinv = _dinv_of(degr[...])
    h = jnp.concatenate([sar[...] + mar[...], sbr[...] + mbr[...]], axis=1)
    h = _gelu(h * dinv + br[...])
    mt = jnp.dot(h, wr[...], preferred_element_type=_f32) * dinv
    oa[...] = mt[:, :HH]
    ob[...] = mt[:, HH:]


_tc_mid = pl.pallas_call(
    _tc_mid_body,
    grid=(GRID,),
    in_specs=[
        pl.BlockSpec((BLK, HH), lambda i: (i, 0)),
        pl.BlockSpec((BLK, HH), lambda i: (i, 0)),
        pl.BlockSpec((BLK, HH), lambda i: (i, 0)),
        pl.BlockSpec((BLK, HH), lambda i: (i, 0)),
        pl.BlockSpec((BLK, 8), lambda i: (i, 0)),
        pl.BlockSpec((1, H), lambda i: (0, 0)),
        pl.BlockSpec((H, H), lambda i: (0, 0)),
    ],
    out_specs=[
        pl.BlockSpec((BLK, HH), lambda i: (i, 0)),
        pl.BlockSpec((BLK, HH), lambda i: (i, 0)),
    ],
    out_shape=[
        jax.ShapeDtypeStruct((N, HH), _f32),
        jax.ShapeDtypeStruct((N, HH), _f32),
    ],
)


def _tc_final_body(sar, sbr, mar, mbr, degr, br, batchr, wfr, bfr, outr, gacc):
    i = pl.program_id(0)
    dinv = _dinv_of(degr[...])
    h = jnp.concatenate([sar[...] + mar[...], sbr[...] + mbr[...]], axis=1)
    h = _gelu(h * dinv + br[...])
    bvec = batchr[0, 0, :]
    onehot = (bvec[:, None] ==
              lax.broadcasted_iota(_i32, (BLK, G), 1)).astype(_f32)
    contrib = lax.dot_general(onehot, h, (((0,), (0,)), ((), ())),
                              preferred_element_type=_f32)

    @pl.when(i == 0)
    def _first():
        gacc[...] = contrib

    @pl.when(i > 0)
    def _rest():
        gacc[...] = gacc[...] + contrib

    @pl.when(i == GRID - 1)
    def _emit():
        outr[...] = _gelu(
            jnp.dot(gacc[...], wfr[...], preferred_element_type=_f32) + bfr[...])


_tc_final = pl.pallas_call(
    _tc_final_body,
    grid=(GRID,),
    in_specs=[
        pl.BlockSpec((BLK, HH), lambda i: (i, 0)),
        pl.BlockSpec((BLK, HH), lambda i: (i, 0)),
        pl.BlockSpec((BLK, HH), lambda i: (i, 0)),
        pl.BlockSpec((BLK, HH), lambda i: (i, 0)),
        pl.BlockSpec((BLK, 8), lambda i: (i, 0)),
        pl.BlockSpec((1, H), lambda i: (0, 0)),
        pl.BlockSpec((1, 1, BLK), lambda i: (i, 0, 0)),
        pl.BlockSpec((H, 128), lambda i: (0, 0)),
        pl.BlockSpec((1, 128), lambda i: (0, 0)),
    ],
    out_specs=pl.BlockSpec((G, 128), lambda i: (0, 0)),
    out_shape=jax.ShapeDtypeStruct((G, 128), _f32),
    scratch_shapes=[pltpu.VMEM((G, H), _f32)],
)


@jax.jit
def kernel(x, edge_index, batch, emb, W1, b1, W2, b2, W3, b3, Wf, bf):
    row2d = edge_index[0].astype(_i32).reshape(NCHUNK, CH)
    col2d = edge_index[1].astype(_i32).reshape(NCHUNK, CH)
    nidx = x[:, -1].astype(_i32)
    nidx2d = jnp.concatenate(
        [nidx, jnp.zeros((NPADN - N,), _i32)]).reshape(NIDXCH, CH)
    feats = x[:, :DIN]
    batch3 = batch.astype(_i32).reshape(GRID, 1, BLK)

    zeros128 = jnp.zeros((N, HH), _f32)
    embp = jnp.pad(emb, ((0, 0), (0, HH - EMBD)))

    dega, degb, embvp = _sc_pre(col2d, nidx2d, embp, jnp.zeros((NHIST,), _f32))
    embv = embvp[:N, :EMBD]
    deg8 = (dega[:N] + degb[:N])[:, None] * jnp.ones((1, 8), _f32)

    m1a, m1b = _tc_layer1(feats, embv, W1[:DIN], W1[DIN:], deg8)
    s1a, s1b = _sc_scatter(m1a, m1b, row2d, col2d, zeros128)
    m2a, m2b = _tc_mid(s1a, s1b, m1a, m1b, deg8, b1.reshape(1, H), W2)
    s2a, s2b = _sc_scatter(m2a, m2b, row2d, col2d, zeros128)
    m3a, m3b = _tc_mid(s2a, s2b, m2a, m2b, deg8, b2.reshape(1, H), W3)
    s3a, s3b = _sc_scatter(m3a, m3b, row2d, col2d, zeros128)

    wfp = jnp.pad(Wf, ((0, 0), (0, 128 - OUTD)))
    bfp = jnp.pad(bf, (0, 128 - OUTD)).reshape(1, 128)
    out128 = _tc_final(s3a, s3b, m3a, m3b, deg8, b3.reshape(1, H),
                       batch3, wfp, bfp)
    return out128[:, :OUTD]


# speedup vs baseline: 1.0388x; 1.0057x over previous
import jax
import jax.numpy as jnp
from jax import lax
from jax.experimental import pallas as pl
from jax.experimental.pallas import tpu as pltpu
from jax.experimental.pallas import tpu_sc as plsc

N = 10000
E = 320000
DIN = 128
EMBD = 64
H = 256
HH = 128
VOCAB = 1000
OUTD = 16
G = 64
NS = 16

CH = 128
NCHUNK = E // CH
UNR = 12
FULL = (NCHUNK // NS) // UNR
REM = NCHUNK - FULL * UNR * NS

NIDXCH = 80
NPADN = NIDXCH * CH

RPT = 624
TAILOFF = RPT * NS
TAIL = N - TAILOFF

BLK = 1000
GRID = N // BLK

_f32 = jnp.float32
_i32 = jnp.int32

_mesh = plsc.VectorSubcoreMesh(core_axis_name="c", subcore_axis_name="s")


NHIST = 10240
HSL = NHIST // NS


def _sc_pre_body(colch, nidx2d, embp, zeros1d,
                 deg, degb, embv,
                 idxc0, idxc1,
                 idxe_v, idxe2, erows, erows2, hist, mslab, msum,
                 hshared,
                 semc0, semc1, semg0, semg1, semo0, semo1, sem_x):
    c = lax.axis_index("c")
    s = lax.axis_index("s")

    pltpu.sync_copy(zeros1d, hist)

    @pl.when(c == 1)
    def _emb():
        NE = NIDXCH // NS
        s5 = s * NE
        ebuf = [idxe_v, idxe2]
        erws = [erows, erows2]
        semg = [semg0, semg1]
        semo = [semo0, semo1]
        g = {}
        o = {}
        pltpu.async_copy(nidx2d.at[s5], ebuf[0], sem_x).wait()
        g[0] = pltpu.async_copy(embp.at[ebuf[0]], erws[0], semg[0])
        for j in range(NE):
            if j + 1 < NE:
                pltpu.async_copy(
                    nidx2d.at[s5 + j + 1], ebuf[(j + 1) % 2], sem_x).wait()
            if j >= 1:
                o[j - 1].wait()
            if j + 1 < NE:
                g[j + 1] = pltpu.async_copy(
                    embp.at[ebuf[(j + 1) % 2]], erws[(j + 1) % 2],
                    semg[(j + 1) % 2])
            g[j].wait()
            o[j] = pltpu.async_copy(
                erws[j % 2], embv.at[pl.ds((s5 + j) * CH, CH)], semo[j % 2])
        o[NE - 1].wait()

    cbuf = [idxc0, idxc1]
    semc = [semc0, semc1]
    base_d = c * (NCHUNK // 2)
    UD = 6
    ones16 = jnp.ones((16,), _f32)

    def _hist_chunk(buf):
        for kk in range(CH // 16):
            idx16 = buf[pl.ds(kk * 16, 16)]
            plsc.addupdate_scatter(hist, [idx16], ones16)

    def dbody(i, _):
        def t(j):
            return base_d + s + NS * (UD * i + j)

        ldc = {}
        for j in range(2):
            ldc[j] = pltpu.async_copy(colch.at[t(j)], cbuf[j], semc[j])
        for j in range(UD):
            ldc[j].wait()
            if j + 2 < UD:
                ldc[j + 2] = pltpu.async_copy(
                    colch.at[t(j + 2)], cbuf[j % 2], semc[j % 2])
            _hist_chunk(cbuf[j % 2])
        return 0

    lax.fori_loop(0, (NCHUNK // 2 - 2) // (UD * NS), dbody, 0)

    @pl.when(s < 2)
    def _extra():
        textra = base_d + NCHUNK // 2 - 2 + s
        pltpu.async_copy(colch.at[textra], idxc0, semc0).wait()
        _hist_chunk(idxc0)

    pltpu.sync_copy(hist, hshared.at[s])
    plsc.subcore_barrier()
    sl = pl.ds(s * HSL, HSL)
    pltpu.sync_copy(hshared.at[0, sl], msum)
    for tt in range(1, NS):
        pltpu.sync_copy(hshared.at[tt, sl], mslab)
        for k in range(HSL // 16):
            w = pl.ds(k * 16, 16)
            msum[w] = msum[w] + mslab[w]

    @pl.when(c == 0)
    def _outa():
        pltpu.sync_copy(msum, deg.at[sl])

    @pl.when(c == 1)
    def _outb():
        pltpu.sync_copy(msum, degb.at[sl])


_sc_pre = pl.kernel(
    _sc_pre_body,
    out_type=(jax.ShapeDtypeStruct((NHIST,), _f32),
              jax.ShapeDtypeStruct((NHIST,), _f32),
              jax.ShapeDtypeStruct((NPADN, HH), _f32)),
    mesh=_mesh,
    scratch_types=[
        pltpu.VMEM((CH,), _i32),
        pltpu.VMEM((CH,), _i32),
        pltpu.VMEM((CH,), _i32),
        pltpu.VMEM((CH,), _i32),
        pltpu.VMEM((CH, HH), _f32),
        pltpu.VMEM((CH, HH), _f32),
        pltpu.VMEM((NHIST,), _f32),
        pltpu.VMEM((HSL,), _f32),
        pltpu.VMEM((HSL,), _f32),
        pltpu.VMEM_SHARED((NS, NHIST), _f32),
    ] + [pltpu.SemaphoreType.DMA] * 7,
    compiler_params=pltpu.CompilerParams(needs_layout_passes=False),
)


def _sc_scatter_body(mA, mB, rowch, colch, zeros128,
                     sA, sB,
                     idxr0, idxr1, idxr2, idxr3,
                     idxc0, idxc1, idxc2, idxc3,
                     rows, acc,
                     semr0, semr1, semr2, semr3,
                     semc0, semc1, semc2, semc3,
                     semg0, semg1, semg2,
                     sems0, sems1, sems2, sems3):
    c = lax.axis_index("c")
    s = lax.axis_index("s")
    slab = pl.ds(s * RPT, RPT)
    tslab = pl.ds(TAILOFF, TAIL)

    pltpu.sync_copy(zeros128.at[slab], acc.at[slab])

    @pl.when(s == NS - 1)
    def _init_tail():
        pltpu.sync_copy(zeros128.at[tslab], acc.at[tslab])

    plsc.subcore_barrier()

    rbuf = [idxr0, idxr1, idxr2, idxr3]
    cbuf = [idxc0, idxc1, idxc2, idxc3]
    semr = [semr0, semr1, semr2, semr3]
    semc = [semc0, semc1, semc2, semc3]
    semg = [semg0, semg1, semg2]
    sems = [sems0, sems1, sems2, sems3]

    def _pipeline(tbl):
        def body(i, _):
            def t(j):
                return s + NS * (UNR * i + j)

            ldr = {}
            ldc = {}
            g = {}
            sc = {}
            for j in range(3):
                ldr[j] = pltpu.async_copy(rowch.at[t(j)], rbuf[j], semr[j])
            for j in range(2):
                ldc[j] = pltpu.async_copy(colch.at[t(j)], cbuf[j], semc[j])
            for j in range(2):
                ldr[j].wait()
                g[j] = pltpu.async_copy(
                    tbl.at[rbuf[j]], rows.at[j % 3], semg[j % 3])
            for j in range(UNR):
                g[j].wait()
                ldc[j].wait()
                pltpu.sync_copy(rows.at[j % 3], acc.at[cbuf[j % 4]], add=True)
                if j + 2 < UNR:
                    ldr[j + 2].wait()
                    g[j + 2] = pltpu.async_copy(
                        tbl.at[rbuf[(j + 2) % 4]], rows.at[(j + 2) % 3],
                        semg[(j + 2) % 3])
                if j + 3 < UNR:
                    ldr[j + 3] = pltpu.async_copy(
                        rowch.at[t(j + 3)], rbuf[(j + 3) % 4],
                        semr[(j + 3) % 4])
                if j + 2 < UNR:
                    ldc[j + 2] = pltpu.async_copy(
                        colch.at[t(j + 2)], cbuf[(j + 2) % 4],
                        semc[(j + 2) % 4])
            return 0

        lax.fori_loop(0, FULL, body, 0)

        @pl.when(s < REM)
        def _extra():
            textra = FULL * UNR * NS + s
            pltpu.async_copy(rowch.at[textra], rbuf[0], semr[0]).wait()
            pltpu.async_copy(colch.at[textra], cbuf[0], semc[0]).wait()
            pltpu.async_copy(tbl.at[rbuf[0]], rows.at[0], semg[0]).wait()
            pltpu.sync_copy(rows.at[0], acc.at[cbuf[0]], add=True)

    @pl.when(c == 0)
    def _pa():
        _pipeline(mA)

    @pl.when(c == 1)
    def _pb():
        _pipeline(mB)

    plsc.subcore_barrier()

    @pl.when(c == 0)
    def _oa():
        pltpu.sync_copy(acc.at[slab], sA.at[slab])

        @pl.when(s == NS - 1)
        def _oa_tail():
            pltpu.sync_copy(acc.at[tslab], sA.at[tslab])

    @pl.when(c == 1)
    def _ob():
        pltpu.sync_copy(acc.at[slab], sB.at[slab])

        @pl.when(s == NS - 1)
        def _ob_tail():
            pltpu.sync_copy(acc.at[tslab], sB.at[tslab])


_sc_scatter = pl.kernel(
    _sc_scatter_body,
    out_type=(jax.ShapeDtypeStruct((N, HH), _f32),
              jax.ShapeDtypeStruct((N, HH), _f32)),
    mesh=_mesh,
    scratch_types=[
        pltpu.VMEM((CH,), _i32),
        pltpu.VMEM((CH,), _i32),
        pltpu.VMEM((CH,), _i32),
        pltpu.VMEM((CH,), _i32),
        pltpu.VMEM((CH,), _i32),
        pltpu.VMEM((CH,), _i32),
        pltpu.VMEM((CH,), _i32),
        pltpu.VMEM((CH,), _i32),
        pltpu.VMEM((3, CH, HH), _f32),
        pltpu.VMEM_SHARED((N, HH), _f32),
    ] + [pltpu.SemaphoreType.DMA] * 15,
)


def _dinv_of(degr):
    return lax.rsqrt(degr[:, 0:1] + 1.0)


def _gelu(v):
    return 0.5 * v * (1.0 + lax.erf(v * 0.7071067811865476))


def _tc_layer1_body(featsr, embvr, w1ar, w1br, degr, oa, ob):
    dinv = _dinv_of(degr[...])
    m = jnp.dot(featsr[...], w1ar[...], preferred_element_type=_f32)
    m = m + jnp.dot(embvr[...], w1br[...], preferred_element_type=_f32)
    mt = m * dinv
    oa[...] = mt[:, :HH]
    ob[...] = mt[:, HH:]


_tc_layer1 = pl.pallas_call(
    _tc_layer1_body,
    grid=(GRID,),
    in_specs=[
        pl.BlockSpec((BLK, DIN), lambda i: (i, 0)),
        pl.BlockSpec((BLK, EMBD), lambda i: (i, 0)),
        pl.BlockSpec((DIN, H), lambda i: (0, 0)),
        pl.BlockSpec((EMBD, H), lambda i: (0, 0)),
        pl.BlockSpec((BLK, 8), lambda i: (i, 0)),
    ],
    out_specs=[
        pl.BlockSpec((BLK, HH), lambda i: (i, 0)),
        pl.BlockSpec((BLK, HH), lambda i: (i, 0)),
    ],
    out_shape=[
        jax.ShapeDtypeStruct((N, HH), _f32),
        jax.ShapeDtypeStruct((N, HH), _f32),
    ],
)


def _tc_mid_body(sar, sbr, mar, mbr, degr, br, wr, oa, ob):
    dinv = _dinv_of(degr[...])
    h = jnp.concatenate([sar[...] + mar[...], sbr[...] + mbr[...]], axis=1)
    h = _gelu(h * dinv + br[...])
    mt = jnp.dot(h, wr[...], preferred_element_type=_f32) * dinv
    oa[...] = mt[:, :HH]
    ob[...] = mt[:, HH:]


_tc_mid = pl.pallas_call(
    _tc_mid_body,
    grid=(GRID,),
    in_specs=[
        pl.BlockSpec((BLK, HH), lambda i: (i, 0)),
        pl.BlockSpec((BLK, HH), lambda i: (i, 0)),
        pl.BlockSpec((BLK, HH), lambda i: (i, 0)),
        pl.BlockSpec((BLK, HH), lambda i: (i, 0)),
        pl.BlockSpec((BLK, 8), lambda i: (i, 0)),
        pl.BlockSpec((1, H), lambda i: (0, 0)),
        pl.BlockSpec((H, H), lambda i: (0, 0)),
    ],
    out_specs=[
        pl.BlockSpec((BLK, HH), lambda i: (i, 0)),
        pl.BlockSpec((BLK, HH), lambda i: (i, 0)),
    ],
    out_shape=[
        jax.ShapeDtypeStruct((N, HH), _f32),
        jax.ShapeDtypeStruct((N, HH), _f32),
    ],
)


def _tc_final_body(sar, sbr, mar, mbr, degr, br, batchr, wfr, bfr, outr, gacc):
    i = pl.program_id(0)
    dinv = _dinv_of(degr[...])
    h = jnp.concatenate([sar[...] + mar[...], sbr[...] + mbr[...]], axis=1)
    h = _gelu(h * dinv + br[...])
    bvec = batchr[0, 0, :]
    onehot = (bvec[:, None] ==
              lax.broadcasted_iota(_i32, (BLK, G), 1)).astype(_f32)
    contrib = lax.dot_general(onehot, h, (((0,), (0,)), ((), ())),
                              preferred_element_type=_f32)

    @pl.when(i == 0)
    def _first():
        gacc[...] = contrib

    @pl.when(i > 0)
    def _rest():
        gacc[...] = gacc[...] + contrib

    @pl.when(i == GRID - 1)
    def _emit():
        outr[...] = _gelu(
            jnp.dot(gacc[...], wfr[...], preferred_element_type=_f32) + bfr[...])


_tc_final = pl.pallas_call(
    _tc_final_body,
    grid=(GRID,),
    in_specs=[
        pl.BlockSpec((BLK, HH), lambda i: (i, 0)),
        pl.BlockSpec((BLK, HH), lambda i: (i, 0)),
        pl.BlockSpec((BLK, HH), lambda i: (i, 0)),
        pl.BlockSpec((BLK, HH), lambda i: (i, 0)),
        pl.BlockSpec((BLK, 8), lambda i: (i, 0)),
        pl.BlockSpec((1, H), lambda i: (0, 0)),
        pl.BlockSpec((1, 1, BLK), lambda i: (i, 0, 0)),
        pl.BlockSpec((H, 128), lambda i: (0, 0)),
        pl.BlockSpec((1, 128), lambda i: (0, 0)),
    ],
    out_specs=pl.BlockSpec((G, 128), lambda i: (0, 0)),
    out_shape=jax.ShapeDtypeStruct((G, 128), _f32),
    scratch_shapes=[pltpu.VMEM((G, H), _f32)],
)


@jax.jit
def kernel(x, edge_index, batch, emb, W1, b1, W2, b2, W3, b3, Wf, bf):
    row2d = edge_index[0].astype(_i32).reshape(NCHUNK, CH)
    col2d = edge_index[1].astype(_i32).reshape(NCHUNK, CH)
    nidx = x[:, -1].astype(_i32)
    nidx2d = jnp.concatenate(
        [nidx, jnp.zeros((NPADN - N,), _i32)]).reshape(NIDXCH, CH)
    feats = x[:, :DIN]
    batch3 = batch.astype(_i32).reshape(GRID, 1, BLK)

    zeros128 = jnp.zeros((N, HH), _f32)
    embp = jnp.pad(emb, ((0, 0), (0, HH - EMBD)))

    dega, degb, embvp = _sc_pre(col2d, nidx2d, embp, jnp.zeros((NHIST,), _f32))
    embv = embvp[:N, :EMBD]
    deg8 = (dega[:N] + degb[:N])[:, None] * jnp.ones((1, 8), _f32)

    m1a, m1b = _tc_layer1(feats, embv, W1[:DIN], W1[DIN:], deg8)
    s1a, s1b = _sc_scatter(m1a, m1b, row2d, col2d, zeros128)
    m2a, m2b = _tc_mid(s1a, s1b, m1a, m1b, deg8, b1.reshape(1, H), W2)
    s2a, s2b = _sc_scatter(m2a, m2b, row2d, col2d, zeros128)
    m3a, m3b = _tc_mid(s2a, s2b, m2a, m2b, deg8, b2.reshape(1, H), W3)
    s3a, s3b = _sc_scatter(m3a, m3b, row2d, col2d, zeros128)

    wfp = jnp.pad(Wf, ((0, 0), (0, 128 - OUTD)))
    bfp = jnp.pad(bf, (0, 128 - OUTD)).reshape(1, 128)
    out128 = _tc_final(s3a, s3b, m3a, m3b, deg8, b3.reshape(1, H),
                       batch3, wfp, bfp)
    return out128[:, :OUTD]
